# Initial kernel scaffold; baseline (speedup 1.0000x reference)
import jax
import jax.numpy as jnp
from jax.experimental import pallas as pl
from jax.experimental.pallas import tpu as pltpu

B = 16384
NUM_CLASSES = 100
WORDS_PER_CLASS = 3
TOPK = 5
VOCAB = 100000
GLOVE_D = 300
FEAT = 1236
D_OUT = 1024

BLK = 256


def _gather_body(idx_ref, w2v_ref, table_ref):
    i = pl.program_id(0)

    @pl.when(i % WORDS_PER_CLASS == 0)
    def _():
        table_ref[...] = jnp.zeros_like(table_ref)

    w = idx_ref[i]
    scale = jnp.where(w == 0, 0.0, 1.0 / WORDS_PER_CLASS).astype(jnp.float32)
    table_ref[...] += w2v_ref[...] * scale


def _main_body(ce_ref, lf_ref, table_ref, w1_ref, w2_ref, b_ref, out_ref,
               cls_out_ref):
    i = pl.program_id(0)

    @pl.when(i == 0)
    def _():
        cls_out_ref[...] = (
            jnp.dot(table_ref[...], w2_ref[...],
                    preferred_element_type=jnp.float32)
            + b_ref[...]
        )

    ce = ce_ref[...]
    iota = jax.lax.broadcasted_iota(jnp.int32, (BLK, NUM_CLASSES), 1)
    counts = jnp.zeros((BLK, NUM_CLASSES), jnp.float32)
    for k in range(TOPK):
        counts += (ce[:, k][:, None] == iota).astype(jnp.float32)
    counts = counts * (1.0 / TOPK)
    out_ref[...] = (
        jnp.dot(lf_ref[...], w1_ref[...], preferred_element_type=jnp.float32)
        + jnp.dot(counts, cls_out_ref[...], preferred_element_type=jnp.float32)
    )


def kernel(layers_feature, classes_embed, class_word_indices, word2vec, W, b):
    idx_flat = class_word_indices.reshape(-1)

    table = pl.pallas_call(
        _gather_body,
        grid_spec=pltpu.PrefetchScalarGridSpec(
            num_scalar_prefetch=1,
            grid=(NUM_CLASSES * WORDS_PER_CLASS,),
            in_specs=[
                pl.BlockSpec((1, GLOVE_D), lambda i, idx_ref: (idx_ref[i], 0)),
            ],
            out_specs=pl.BlockSpec(
                (1, GLOVE_D), lambda i, idx_ref: (i // WORDS_PER_CLASS, 0)),
        ),
        out_shape=jax.ShapeDtypeStruct((NUM_CLASSES, GLOVE_D), jnp.float32),
    )(idx_flat, word2vec)

    W1 = W[:FEAT]
    W2 = W[FEAT:]
    b2 = b.reshape(1, D_OUT)

    out = pl.pallas_call(
        _main_body,
        grid=(B // BLK,),
        in_specs=[
            pl.BlockSpec((BLK, TOPK), lambda i: (i, 0)),
            pl.BlockSpec((BLK, FEAT), lambda i: (i, 0)),
            pl.BlockSpec((NUM_CLASSES, GLOVE_D), lambda i: (0, 0)),
            pl.BlockSpec((FEAT, D_OUT), lambda i: (0, 0)),
            pl.BlockSpec((GLOVE_D, D_OUT), lambda i: (0, 0)),
            pl.BlockSpec((1, D_OUT), lambda i: (0, 0)),
        ],
        out_specs=pl.BlockSpec((BLK, D_OUT), lambda i: (i, 0)),
        out_shape=jax.ShapeDtypeStruct((B, D_OUT), jnp.float32),
        scratch_shapes=[pltpu.VMEM((NUM_CLASSES, D_OUT), jnp.float32)],
    )(classes_embed, layers_feature, table, W1, W2, b2)
    return out


# speedup vs baseline: 6.1086x; 6.1086x over previous
import jax
import jax.numpy as jnp
from jax.experimental import pallas as pl
from jax.experimental.pallas import tpu as pltpu

B = 16384
NUM_CLASSES = 100
WORDS_PER_CLASS = 3
TOPK = 5
VOCAB = 100000
GLOVE_D = 300
FEAT = 1236
D_OUT = 1024

BLK = 256


def _gather_body(idx_ref, w2v_ref, table_ref, buf_ref, sem_ref):
    def issue(c, u):
        for k in range(WORDS_PER_CLASS):
            w = idx_ref[c * WORDS_PER_CLASS + k]
            pltpu.make_async_copy(
                w2v_ref.at[pl.ds(w, 1), :], buf_ref.at[u, k], sem_ref.at[u, k]
            ).start()

    issue(0, 0)
    issue(1, 1)

    def body(i, carry):
        for u in range(2):
            c = 2 * i + u
            acc = jnp.zeros((1, GLOVE_D), jnp.float32)
            for k in range(WORDS_PER_CLASS):
                pltpu.make_async_copy(
                    w2v_ref.at[pl.ds(0, 1), :], buf_ref.at[u, k],
                    sem_ref.at[u, k]).wait()
                w = idx_ref[c * WORDS_PER_CLASS + k]
                mk = jnp.where(w == 0, 0.0, 1.0 / WORDS_PER_CLASS)
                acc = acc + buf_ref[u, k] * mk
            table_ref[pl.ds(c, 1), :] = acc

            @pl.when(c + 2 < NUM_CLASSES)
            def _():
                issue(c + 2, u)

        return carry

    jax.lax.fori_loop(0, NUM_CLASSES // 2, body, 0)


def _main_body(ce_ref, lf_ref, table_ref, w1_ref, w2_ref, b_ref, out_ref,
               cls_out_ref):
    i = pl.program_id(0)

    @pl.when(i == 0)
    def _():
        cls_out_ref[...] = (
            jnp.dot(table_ref[...], w2_ref[...],
                    preferred_element_type=jnp.float32)
            + b_ref[...]
        )

    ce = ce_ref[...]
    iota = jax.lax.broadcasted_iota(jnp.int32, (BLK, NUM_CLASSES), 1)
    counts = jnp.zeros((BLK, NUM_CLASSES), jnp.float32)
    for k in range(TOPK):
        counts += (ce[:, k][:, None] == iota).astype(jnp.float32)
    counts = counts * (1.0 / TOPK)
    out_ref[...] = (
        jnp.dot(lf_ref[...], w1_ref[...], preferred_element_type=jnp.float32)
        + jnp.dot(counts, cls_out_ref[...], preferred_element_type=jnp.float32)
    )


def kernel(layers_feature, classes_embed, class_word_indices, word2vec, W, b):
    idx_flat = class_word_indices.reshape(-1)

    table = pl.pallas_call(
        _gather_body,
        in_specs=[
            pl.BlockSpec(memory_space=pltpu.MemorySpace.SMEM),
            pl.BlockSpec(memory_space=pltpu.MemorySpace.HBM),
        ],
        out_specs=pl.BlockSpec(memory_space=pltpu.MemorySpace.VMEM),
        out_shape=jax.ShapeDtypeStruct((NUM_CLASSES, GLOVE_D), jnp.float32),
        scratch_shapes=[
            pltpu.VMEM((2, WORDS_PER_CLASS, 1, GLOVE_D), jnp.float32),
            pltpu.SemaphoreType.DMA((2, WORDS_PER_CLASS)),
        ],
    )(idx_flat, word2vec)

    W1 = W[:FEAT]
    W2 = W[FEAT:]
    b2 = b.reshape(1, D_OUT)

    out = pl.pallas_call(
        _main_body,
        grid=(B // BLK,),
        in_specs=[
            pl.BlockSpec((BLK, TOPK), lambda i: (i, 0)),
            pl.BlockSpec((BLK, FEAT), lambda i: (i, 0)),
            pl.BlockSpec((NUM_CLASSES, GLOVE_D), lambda i: (0, 0)),
            pl.BlockSpec((FEAT, D_OUT), lambda i: (0, 0)),
            pl.BlockSpec((GLOVE_D, D_OUT), lambda i: (0, 0)),
            pl.BlockSpec((1, D_OUT), lambda i: (0, 0)),
        ],
        out_specs=pl.BlockSpec((BLK, D_OUT), lambda i: (i, 0)),
        out_shape=jax.ShapeDtypeStruct((B, D_OUT), jnp.float32),
        scratch_shapes=[pltpu.VMEM((NUM_CLASSES, D_OUT), jnp.float32)],
    )(classes_embed, layers_feature, table, W1, W2, b2)
    return out


# speedup vs baseline: 6.6924x; 1.0956x over previous
import jax
import jax.numpy as jnp
from jax.experimental import pallas as pl
from jax.experimental.pallas import tpu as pltpu
from jax.experimental.pallas import tpu_sc as plsc

B = 16384
NUM_CLASSES = 100
WORDS_PER_CLASS = 3
TOPK = 5
VOCAB = 100000
GLOVE_D = 300
FEAT = 1236
D_OUT = 1024

BLK = 256


NW = 16
KPW = 24
ACC_ROWS = 128
TRASH = 127


def _sc_gather_body(idx_hbm, tgt_hbm, w2v_hbm, zeros_hbm, out_hbm,
                    idx_v, tgt_v, rows_v, acc, sem):
    c = jax.lax.axis_index("c")
    s = jax.lax.axis_index("s")

    @pl.when(c == 0)
    def _():
        pltpu.sync_copy(zeros_hbm, acc.at[pl.ds(s * 8, 8)])
        plsc.subcore_barrier()
        pltpu.sync_copy(idx_hbm.at[s], idx_v)
        pltpu.sync_copy(tgt_hbm.at[s], tgt_v)
        pltpu.async_copy(w2v_hbm.at[idx_v], rows_v, sem).wait()
        pltpu.sync_copy(rows_v, acc.at[tgt_v], add=True)
        plsc.subcore_barrier()

        @pl.when(s < NUM_CLASSES // 8)
        def _():
            pltpu.sync_copy(acc.at[pl.ds(s * 8, 8)], out_hbm.at[pl.ds(s * 8, 8)])

        @pl.when(s == NUM_CLASSES // 8)
        def _():
            pltpu.sync_copy(acc.at[pl.ds(96, NUM_CLASSES - 96)],
                            out_hbm.at[pl.ds(96, NUM_CLASSES - 96)])


def _gather_body(idx_ref, cwi_ref, w2v_ref, table_ref, buf_ref, sem_ref):
    for c in range(NUM_CLASSES):
        for k in range(WORDS_PER_CLASS):
            w = idx_ref[c * WORDS_PER_CLASS + k]
            pltpu.make_async_copy(
                w2v_ref.at[pl.ds(w, 1), :], buf_ref.at[k, pl.ds(c, 1), :],
                sem_ref.at[k, c]).start()
    for c in range(NUM_CLASSES):
        for k in range(WORDS_PER_CLASS):
            pltpu.make_async_copy(
                w2v_ref.at[pl.ds(0, 1), :], buf_ref.at[k, pl.ds(c, 1), :],
                sem_ref.at[k, c]).wait()
    m = (cwi_ref[...] != 0).astype(jnp.float32) * (1.0 / WORDS_PER_CLASS)
    acc = buf_ref[0] * m[:, 0:1]
    acc += buf_ref[1] * m[:, 1:2]
    acc += buf_ref[2] * m[:, 2:3]
    table_ref[...] = acc


def _main_body(ce_ref, lf_ref, table_ref, w1_ref, w2_ref, b_ref,
               out_ref, cls_out_ref):
    i = pl.program_id(0)

    @pl.when(i == 0)
    def _():
        cls_out_ref[...] = (
            jnp.dot(table_ref[...], w2_ref[...],
                    preferred_element_type=jnp.float32)
            + b_ref[...]
        )

    ce = ce_ref[...]
    iota = jax.lax.broadcasted_iota(jnp.int32, (BLK, NUM_CLASSES), 1)
    counts = jnp.zeros((BLK, NUM_CLASSES), jnp.float32)
    for k in range(TOPK):
        counts += (ce[:, k][:, None] == iota).astype(jnp.float32)
    counts = counts * (1.0 / TOPK)
    out_ref[...] = (
        jnp.dot(lf_ref[...], w1_ref[...], preferred_element_type=jnp.float32)
        + jnp.dot(counts, cls_out_ref[...], preferred_element_type=jnp.float32)
    )


def kernel(layers_feature, classes_embed, class_word_indices, word2vec, W, b):
    idx_flat = class_word_indices.reshape(-1)

    table = pl.pallas_call(
        _gather_body,
        in_specs=[
            pl.BlockSpec(memory_space=pltpu.MemorySpace.SMEM),
            pl.BlockSpec(memory_space=pltpu.MemorySpace.VMEM),
            pl.BlockSpec(memory_space=pltpu.MemorySpace.HBM),
        ],
        out_specs=pl.BlockSpec(memory_space=pltpu.MemorySpace.VMEM),
        out_shape=jax.ShapeDtypeStruct((NUM_CLASSES, GLOVE_D), jnp.float32),
        scratch_shapes=[
            pltpu.VMEM((WORDS_PER_CLASS, NUM_CLASSES, GLOVE_D), jnp.float32),
            pltpu.SemaphoreType.DMA((WORDS_PER_CLASS, NUM_CLASSES)),
        ],
    )(idx_flat, class_word_indices, word2vec)

    W1 = W[:FEAT]
    W2 = W[FEAT:]
    b2 = b.reshape(1, D_OUT)

    out = pl.pallas_call(
        _main_body,
        grid=(B // BLK,),
        in_specs=[
            pl.BlockSpec((BLK, TOPK), lambda i: (i, 0)),
            pl.BlockSpec((BLK, FEAT), lambda i: (i, 0)),
            pl.BlockSpec((NUM_CLASSES, GLOVE_D), lambda i: (0, 0)),
            pl.BlockSpec((FEAT, D_OUT), lambda i: (0, 0)),
            pl.BlockSpec((GLOVE_D, D_OUT), lambda i: (0, 0)),
            pl.BlockSpec((1, D_OUT), lambda i: (0, 0)),
        ],
        out_specs=pl.BlockSpec((BLK, D_OUT), lambda i: (i, 0)),
        out_shape=jax.ShapeDtypeStruct((B, D_OUT), jnp.float32),
        scratch_shapes=[pltpu.VMEM((NUM_CLASSES, D_OUT), jnp.float32)],
    )(classes_embed, layers_feature, table, W1, W2, b2)
    return out


# speedup vs baseline: 7.1179x; 1.0636x over previous
import jax
import jax.numpy as jnp
from jax.experimental import pallas as pl
from jax.experimental.pallas import tpu as pltpu
from jax.experimental.pallas import tpu_sc as plsc

B = 16384
NUM_CLASSES = 100
WORDS_PER_CLASS = 3
TOPK = 5
VOCAB = 100000
GLOVE_D = 300
FEAT = 1236
D_OUT = 1024

BLK = 512


NW = 16
KPW = 24
ACC_ROWS = 128
TRASH = 127


def _sc_gather_body(idx_hbm, tgt_hbm, w2v_hbm, zeros_hbm, out_hbm,
                    idx_v, tgt_v, rows_v, acc, sem):
    c = jax.lax.axis_index("c")
    s = jax.lax.axis_index("s")

    @pl.when(c == 0)
    def _():
        pltpu.sync_copy(zeros_hbm, acc.at[pl.ds(s * 8, 8)])
        plsc.subcore_barrier()
        pltpu.sync_copy(idx_hbm.at[s], idx_v)
        pltpu.sync_copy(tgt_hbm.at[s], tgt_v)
        pltpu.async_copy(w2v_hbm.at[idx_v], rows_v, sem).wait()
        pltpu.sync_copy(rows_v, acc.at[tgt_v], add=True)
        plsc.subcore_barrier()

        @pl.when(s < NUM_CLASSES // 8)
        def _():
            pltpu.sync_copy(acc.at[pl.ds(s * 8, 8)], out_hbm.at[pl.ds(s * 8, 8)])

        @pl.when(s == NUM_CLASSES // 8)
        def _():
            pltpu.sync_copy(acc.at[pl.ds(96, NUM_CLASSES - 96)],
                            out_hbm.at[pl.ds(96, NUM_CLASSES - 96)])


def _gather_body(idx_ref, cwi_ref, w2v_ref, table_ref, buf_ref, sem_ref):
    for c in range(NUM_CLASSES):
        for k in range(WORDS_PER_CLASS):
            w = idx_ref[c * WORDS_PER_CLASS + k]
            pltpu.make_async_copy(
                w2v_ref.at[pl.ds(w, 1), :], buf_ref.at[k, pl.ds(c, 1), :],
                sem_ref.at[k, c]).start()
    for c in range(NUM_CLASSES):
        for k in range(WORDS_PER_CLASS):
            pltpu.make_async_copy(
                w2v_ref.at[pl.ds(0, 1), :], buf_ref.at[k, pl.ds(c, 1), :],
                sem_ref.at[k, c]).wait()
    m = (cwi_ref[...] != 0).astype(jnp.float32) * (1.0 / WORDS_PER_CLASS)
    acc = buf_ref[0] * m[:, 0:1]
    acc += buf_ref[1] * m[:, 1:2]
    acc += buf_ref[2] * m[:, 2:3]
    table_ref[...] = acc


def _main_body(ce_ref, lf_ref, table_ref, w1_ref, w2_ref, b_ref,
               out_ref, cls_out_ref):
    i = pl.program_id(0)

    @pl.when(i == 0)
    def _():
        cls_out_ref[...] = (
            jnp.dot(table_ref[...], w2_ref[...],
                    preferred_element_type=jnp.float32)
            + b_ref[...]
        )

    ce = ce_ref[...]
    iota = jax.lax.broadcasted_iota(jnp.int32, (BLK, NUM_CLASSES), 1)
    counts = jnp.zeros((BLK, NUM_CLASSES), jnp.float32)
    for k in range(TOPK):
        counts += (ce[:, k][:, None] == iota).astype(jnp.float32)
    counts = counts * (1.0 / TOPK)
    out_ref[...] = (
        jnp.dot(lf_ref[...], w1_ref[...], preferred_element_type=jnp.float32)
        + jnp.dot(counts, cls_out_ref[...], preferred_element_type=jnp.float32)
    )


def kernel(layers_feature, classes_embed, class_word_indices, word2vec, W, b):
    idx_flat = class_word_indices.reshape(-1)

    table = pl.pallas_call(
        _gather_body,
        in_specs=[
            pl.BlockSpec(memory_space=pltpu.MemorySpace.SMEM),
            pl.BlockSpec(memory_space=pltpu.MemorySpace.VMEM),
            pl.BlockSpec(memory_space=pltpu.MemorySpace.HBM),
        ],
        out_specs=pl.BlockSpec(memory_space=pltpu.MemorySpace.VMEM),
        out_shape=jax.ShapeDtypeStruct((NUM_CLASSES, GLOVE_D), jnp.float32),
        scratch_shapes=[
            pltpu.VMEM((WORDS_PER_CLASS, NUM_CLASSES, GLOVE_D), jnp.float32),
            pltpu.SemaphoreType.DMA((WORDS_PER_CLASS, NUM_CLASSES)),
        ],
    )(idx_flat, class_word_indices, word2vec)

    W1 = W[:FEAT]
    W2 = W[FEAT:]
    b2 = b.reshape(1, D_OUT)

    out = pl.pallas_call(
        _main_body,
        grid=(B // BLK,),
        in_specs=[
            pl.BlockSpec((BLK, TOPK), lambda i: (i, 0)),
            pl.BlockSpec((BLK, FEAT), lambda i: (i, 0)),
            pl.BlockSpec((NUM_CLASSES, GLOVE_D), lambda i: (0, 0)),
            pl.BlockSpec((FEAT, D_OUT), lambda i: (0, 0)),
            pl.BlockSpec((GLOVE_D, D_OUT), lambda i: (0, 0)),
            pl.BlockSpec((1, D_OUT), lambda i: (0, 0)),
        ],
        out_specs=pl.BlockSpec((BLK, D_OUT), lambda i: (i, 0)),
        out_shape=jax.ShapeDtypeStruct((B, D_OUT), jnp.float32),
        scratch_shapes=[pltpu.VMEM((NUM_CLASSES, D_OUT), jnp.float32)],
    )(classes_embed, layers_feature, table, W1, W2, b2)
    return out


# speedup vs baseline: 7.2869x; 1.0237x over previous
import jax
import jax.numpy as jnp
from jax.experimental import pallas as pl
from jax.experimental.pallas import tpu as pltpu
from jax.experimental.pallas import tpu_sc as plsc

B = 16384
NUM_CLASSES = 100
WORDS_PER_CLASS = 3
TOPK = 5
VOCAB = 100000
GLOVE_D = 300
FEAT = 1236
D_OUT = 1024

BLK = 1024


NW = 16
KPW = 24
ACC_ROWS = 128
TRASH = 127


def _sc_gather_body(idx_hbm, tgt_hbm, w2v_hbm, zeros_hbm, out_hbm,
                    idx_v, tgt_v, rows_v, acc, sem):
    c = jax.lax.axis_index("c")
    s = jax.lax.axis_index("s")

    @pl.when(c == 0)
    def _():
        pltpu.sync_copy(zeros_hbm, acc.at[pl.ds(s * 8, 8)])
        plsc.subcore_barrier()
        pltpu.sync_copy(idx_hbm.at[s], idx_v)
        pltpu.sync_copy(tgt_hbm.at[s], tgt_v)
        pltpu.async_copy(w2v_hbm.at[idx_v], rows_v, sem).wait()
        pltpu.sync_copy(rows_v, acc.at[tgt_v], add=True)
        plsc.subcore_barrier()

        @pl.when(s < NUM_CLASSES // 8)
        def _():
            pltpu.sync_copy(acc.at[pl.ds(s * 8, 8)], out_hbm.at[pl.ds(s * 8, 8)])

        @pl.when(s == NUM_CLASSES // 8)
        def _():
            pltpu.sync_copy(acc.at[pl.ds(96, NUM_CLASSES - 96)],
                            out_hbm.at[pl.ds(96, NUM_CLASSES - 96)])


def _gather_body(idx_ref, cwi_ref, w2v_ref, table_ref, buf_ref, sem_ref):
    for c in range(NUM_CLASSES):
        for k in range(WORDS_PER_CLASS):
            w = idx_ref[c * WORDS_PER_CLASS + k]
            pltpu.make_async_copy(
                w2v_ref.at[pl.ds(w, 1), :], buf_ref.at[k, pl.ds(c, 1), :],
                sem_ref.at[k, c]).start()
    for c in range(NUM_CLASSES):
        for k in range(WORDS_PER_CLASS):
            pltpu.make_async_copy(
                w2v_ref.at[pl.ds(0, 1), :], buf_ref.at[k, pl.ds(c, 1), :],
                sem_ref.at[k, c]).wait()
    m = (cwi_ref[...] != 0).astype(jnp.float32) * (1.0 / WORDS_PER_CLASS)
    acc = buf_ref[0] * m[:, 0:1]
    acc += buf_ref[1] * m[:, 1:2]
    acc += buf_ref[2] * m[:, 2:3]
    table_ref[...] = acc


def _main_body(ce_ref, lf_ref, table_ref, w1_ref, w2_ref, b_ref,
               out_ref, cls_out_ref):
    i = pl.program_id(0)

    @pl.when(i == 0)
    def _():
        cls_out_ref[...] = (
            jnp.dot(table_ref[...], w2_ref[...],
                    preferred_element_type=jnp.float32)
            + b_ref[...]
        )

    ce = ce_ref[...]
    iota = jax.lax.broadcasted_iota(jnp.int32, (BLK, NUM_CLASSES), 1)
    counts = jnp.zeros((BLK, NUM_CLASSES), jnp.float32)
    for k in range(TOPK):
        counts += (ce[:, k][:, None] == iota).astype(jnp.float32)
    counts = counts * (1.0 / TOPK)
    out_ref[...] = (
        jnp.dot(lf_ref[...], w1_ref[...], preferred_element_type=jnp.float32)
        + jnp.dot(counts, cls_out_ref[...], preferred_element_type=jnp.float32)
    )


def kernel(layers_feature, classes_embed, class_word_indices, word2vec, W, b):
    idx_flat = class_word_indices.reshape(-1)

    table = pl.pallas_call(
        _gather_body,
        in_specs=[
            pl.BlockSpec(memory_space=pltpu.MemorySpace.SMEM),
            pl.BlockSpec(memory_space=pltpu.MemorySpace.VMEM),
            pl.BlockSpec(memory_space=pltpu.MemorySpace.HBM),
        ],
        out_specs=pl.BlockSpec(memory_space=pltpu.MemorySpace.VMEM),
        out_shape=jax.ShapeDtypeStruct((NUM_CLASSES, GLOVE_D), jnp.float32),
        scratch_shapes=[
            pltpu.VMEM((WORDS_PER_CLASS, NUM_CLASSES, GLOVE_D), jnp.float32),
            pltpu.SemaphoreType.DMA((WORDS_PER_CLASS, NUM_CLASSES)),
        ],
    )(idx_flat, class_word_indices, word2vec)

    W1 = W[:FEAT]
    W2 = W[FEAT:]
    b2 = b.reshape(1, D_OUT)

    out = pl.pallas_call(
        _main_body,
        grid=(B // BLK,),
        in_specs=[
            pl.BlockSpec((BLK, TOPK), lambda i: (i, 0)),
            pl.BlockSpec((BLK, FEAT), lambda i: (i, 0)),
            pl.BlockSpec((NUM_CLASSES, GLOVE_D), lambda i: (0, 0)),
            pl.BlockSpec((FEAT, D_OUT), lambda i: (0, 0)),
            pl.BlockSpec((GLOVE_D, D_OUT), lambda i: (0, 0)),
            pl.BlockSpec((1, D_OUT), lambda i: (0, 0)),
        ],
        out_specs=pl.BlockSpec((BLK, D_OUT), lambda i: (i, 0)),
        out_shape=jax.ShapeDtypeStruct((B, D_OUT), jnp.float32),
        scratch_shapes=[pltpu.VMEM((NUM_CLASSES, D_OUT), jnp.float32)],
    )(classes_embed, layers_feature, table, W1, W2, b2)
    return out
